# baseline (speedup 1.0000x reference)
import functools

import jax
import jax.numpy as jnp
from jax import lax
from jax.experimental import pallas as pl
from jax.experimental.pallas import tpu as pltpu
from jax.experimental.pallas import tpu_sc as plsc

F32 = jnp.float32
I32 = jnp.int32

NC = 2
NS = 16
NW = NC * NS
L = 16

CHUNK = 128


def _sc_mesh():
    return plsc.VectorSubcoreMesh(
        core_axis_name="c", subcore_axis_name="s", num_cores=NC, num_subcores=NS
    )


_SC_PARAMS = pltpu.CompilerParams(
    needs_layout_passes=False, use_tc_tiling_on_sc=False
)


def _k1_body(x_ref, wg_ref, a2_ref, h_ref, aa_ref):
    h = jnp.dot(x_ref[...], wg_ref[...], preferred_element_type=F32)
    h_ref[...] = h
    aa_ref[...] = jnp.dot(h, a2_ref[...], preferred_element_type=F32)


def _gat_scalar_body(np_, pw, aa_hbm, src_hbm, dst_hbm, ex_out, den_out,
                     deg_out, aa_v, srcv, dstv, ex_v, den_v, deg_v):
    c = lax.axis_index("c")
    s = lax.axis_index("s")
    wid = c * NS + s

    pltpu.sync_copy(aa_hbm, aa_v)
    wbase = wid * pw
    pltpu.sync_copy(src_hbm.at[pl.ds(wbase, pw)], srcv)
    pltpu.sync_copy(dst_hbm.at[pl.ds(wbase, pw)], dstv)
    zf = jnp.zeros((L,), F32)

    def _zero(i, _):
        den_v[pl.ds(i * L, L)] = zf
        deg_v[pl.ds(i * L, L)] = zf
        return 0
    lax.fori_loop(0, np_ // L, _zero, 0)

    ones_f = jnp.full((L,), 1.0, F32)

    def _grp(g, _):
        sidx = srcv[pl.ds(g * L, L)]
        didx = dstv[pl.ds(g * L, L)]
        a_s = plsc.load_gather(aa_v, [sidx * 2])
        a_d = plsc.load_gather(aa_v, [didx * 2 + 1])
        e = a_s + a_d
        e = jnp.where(e > 0, e, e * 0.2)
        ex = jnp.exp(e)
        ex_v[pl.ds(g * L, L)] = ex
        plsc.addupdate_scatter(den_v, [didx], ex)
        plsc.addupdate_scatter(deg_v, [didx], ones_f)
        return 0

    lax.fori_loop(0, pw // L, _grp, 0)

    pltpu.sync_copy(ex_v, ex_out.at[pl.ds(wbase, pw)])
    pltpu.sync_copy(den_v, den_out.at[wid])
    pltpu.sync_copy(deg_v, deg_out.at[wid])


HC = CHUNK // 2


def _gat_row_body(np_, chunks, h_hbm, ex_hbm, src3_hbm, dst4_hbm, z128_hbm,
                  acc_out, ex_v, src0, src1, dst0, dst1, rows0, rows1,
                  acc_sh, semA, semB, semS):
    c = lax.axis_index("c")
    s = lax.axis_index("s")
    wid = c * NS + s
    rpt = np_ // NS

    pltpu.sync_copy(z128_hbm.at[pl.ds(s * rpt, rpt)], acc_sh.at[pl.ds(s * rpt, rpt)])
    plsc.subcore_barrier()

    wbase = wid * (chunks * CHUNK)
    pltpu.sync_copy(ex_hbm.at[pl.ds(wbase, chunks * CHUNK)], ex_v)

    def _stage(ci0, srcb, dstb):
        pltpu.sync_copy(src3_hbm.at[wid, pl.ds(ci0, 2)], srcb)
        pltpu.sync_copy(dst4_hbm.at[wid, pl.ds(ci0 * 2, 4)], dstb)

    def _scale_half(rows_v, ci, hh):
        base = ci * CHUNK + hh * HC

        def body(g, _):
            ex16 = ex_v[pl.ds(base + g * L, L)]
            for t in range(L):
                w = ex16.at[jnp.full((L,), t, I32)].get(
                    mode="promise_in_bounds")
                i = hh * HC + g * L + t
                for j in range(128 // L):
                    v = rows_v[i, pl.ds(j * L, L)]
                    rows_v[i, pl.ds(j * L, L)] = v * w
            return 0
        lax.fori_loop(0, HC // L, body, 0)

    def _proc(rows, semG, ci, dstb, jp):
        pltpu.make_async_copy(h_hbm.at[pl.ds(0, CHUNK)], rows, semG).wait()
        _scale_half(rows, ci, 0)
        pltpu.async_copy(rows.at[pl.ds(0, HC)], acc_sh.at[dstb.at[2 * jp]],
                         semS, add=True)
        _scale_half(rows, ci, 1)
        pltpu.async_copy(rows.at[pl.ds(HC, HC)], acc_sh.at[dstb.at[2 * jp + 1]],
                         semS, add=True)
        for _ in range(2):
            pltpu.make_async_copy(h_hbm.at[pl.ds(0, HC)],
                                  rows.at[pl.ds(0, HC)], semS).wait()

    _stage(0, src0, dst0)
    pltpu.async_copy(h_hbm.at[src0.at[0]], rows0, semA)

    def _quad(q, _):
        c0 = q * 4
        pltpu.async_copy(h_hbm.at[src0.at[1]], rows1, semB)
        _stage(c0 + 2, src1, dst1)

        _proc(rows0, semA, c0, dst0, 0)
        pltpu.async_copy(h_hbm.at[src1.at[0]], rows0, semA)

        _proc(rows1, semB, c0 + 1, dst0, 1)
        pltpu.async_copy(h_hbm.at[src1.at[1]], rows1, semB)

        @pl.when(c0 + 4 < chunks)
        def _():
            _stage(c0 + 4, src0, dst0)

        _proc(rows0, semA, c0 + 2, dst1, 0)

        @pl.when(c0 + 4 < chunks)
        def _():
            pltpu.async_copy(h_hbm.at[src0.at[0]], rows0, semA)

        _proc(rows1, semB, c0 + 3, dst1, 1)
        return 0

    lax.fori_loop(0, chunks // 4, _quad, 0)

    plsc.subcore_barrier()
    pltpu.sync_copy(acc_sh.at[pl.ds(s * rpt, rpt)], acc_out.at[c, pl.ds(s * rpt, rpt)])


def _k3_body(n, accp_ref, den_ref, deg_ref, bgat_ref, wgcn_ref, hw2_ref,
             dinv_ref):
    srow = accp_ref[0] + accp_ref[1]
    den = jnp.sum(den_ref[...], axis=1, keepdims=True)
    deg = jnp.sum(deg_ref[...], axis=1, keepdims=True)
    h1 = jnp.tanh(srow / (den + 1e-16) + bgat_ref[...])
    hw = jnp.dot(h1, wgcn_ref[...], preferred_element_type=F32)
    blk = hw.shape[0]
    rowi = pl.program_id(0) * blk + lax.broadcasted_iota(I32, (blk, 1), 0)
    dinv = jnp.where((deg > 0) & (rowi < n), lax.rsqrt(deg), 0.0)
    hw2_ref[...] = hw * dinv
    dinv_ref[...] = dinv


def _gcn_edge_body(np_, chunks, hw2_hbm, src3_hbm, dst3_hbm, z64_hbm, acc_out,
                   srcv, dstv, rows0, rows1, rows2, acc_sh, hw2_sh, semA,
                   semB, semC):
    c = lax.axis_index("c")
    s = lax.axis_index("s")
    wid = c * NS + s
    rpt = np_ // NS

    pltpu.sync_copy(z64_hbm.at[pl.ds(s * rpt, rpt)], acc_sh.at[pl.ds(s * rpt, rpt)])
    pltpu.sync_copy(hw2_hbm.at[pl.ds(s * rpt, rpt)], hw2_sh.at[pl.ds(s * rpt, rpt)])
    pltpu.sync_copy(src3_hbm.at[wid], srcv)
    pltpu.sync_copy(dst3_hbm.at[wid], dstv)
    plsc.subcore_barrier()

    bufs = (rows0, rows1, rows2)
    sems = (semA, semB, semC)
    for k in range(3):
        pltpu.async_copy(hw2_sh.at[srcv.at[k]], bufs[k], sems[k])

    def _trip(q, _):
        for k in range(3):
            ci = q * 3 + k
            pltpu.make_async_copy(hw2_hbm.at[pl.ds(0, CHUNK)], bufs[k], sems[k]).wait()
            pltpu.sync_copy(bufs[k], acc_sh.at[dstv.at[ci]], add=True)

            @pl.when(ci + 3 < chunks)
            def _():
                pltpu.async_copy(hw2_sh.at[srcv.at[ci + 3]], bufs[k], sems[k])
        return 0

    lax.fori_loop(0, chunks // 3, _trip, 0)

    plsc.subcore_barrier()
    pltpu.sync_copy(acc_sh.at[pl.ds(s * rpt, rpt)], acc_out.at[c, pl.ds(s * rpt, rpt)])


def _k5_body(acc2p_ref, dinv_ref, bgcn_ref, z_ref):
    z_ref[...] = dinv_ref[...] * (acc2p_ref[0] + acc2p_ref[1]) + bgcn_ref[...]


def _decode_body(np_, chunks, z_hbm, ea_hbm, eb_hbm, out_hbm, eav, ebv,
                 za0, zb0, za1, zb1, dots_v, z_sh, semA, semB):
    c = lax.axis_index("c")
    s = lax.axis_index("s")
    wid = c * NS + s
    iota = lax.iota(I32, L)
    wbase = wid * (chunks * CHUNK)
    rpt = np_ // NS

    pltpu.sync_copy(z_hbm.at[pl.ds(s * rpt, rpt)], z_sh.at[pl.ds(s * rpt, rpt)])
    pltpu.sync_copy(ea_hbm.at[pl.ds(wbase, chunks * CHUNK)], eav)
    pltpu.sync_copy(eb_hbm.at[pl.ds(wbase, chunks * CHUNK)], ebv)
    plsc.subcore_barrier()

    def _dots(za, zb, k):
        def _grp(g, _):
            rows = g * L + iota
            acc = jnp.zeros((L,), F32)
            for j in range(64):
                col = (iota + j) & 63
                acc = acc + (plsc.load_gather(za, [rows, col])
                             * plsc.load_gather(zb, [rows, col]))
            dots_v[pl.ds(k * CHUNK + g * L, L)] = acc
            return 0
        lax.fori_loop(0, CHUNK // L, _grp, 0)

    def _fire(ci, za, zb, sem):
        pltpu.async_copy(z_sh.at[eav.at[pl.ds(ci * CHUNK, CHUNK)]], za, sem)
        pltpu.async_copy(z_sh.at[ebv.at[pl.ds(ci * CHUNK, CHUNK)]], zb, sem)

    def _drain(za, zb, sem):
        pltpu.make_async_copy(z_hbm.at[pl.ds(0, CHUNK)], za, sem).wait()
        pltpu.make_async_copy(z_hbm.at[pl.ds(0, CHUNK)], zb, sem).wait()

    _fire(0, za0, zb0, semA)

    def _pair(p, _):
        ci0 = p * 2
        _fire(ci0 + 1, za1, zb1, semB)

        _drain(za0, zb0, semA)
        _dots(za0, zb0, 0)

        @pl.when(ci0 + 2 < chunks)
        def _():
            _fire(ci0 + 2, za0, zb0, semA)

        _drain(za1, zb1, semB)
        _dots(za1, zb1, 1)
        pltpu.sync_copy(dots_v, out_hbm.at[pl.ds(wbase + ci0 * CHUNK, 2 * CHUNK)])
        return 0

    lax.fori_loop(0, chunks // 2, _pair, 0)


def kernel(x, pos_edge_index, neg_edge_index, W_gat, a_src, a_dst, b_gat,
           W_gcn, b_gcn):
    n, d = x.shape
    h_dim = W_gat.shape[1]
    o_dim = W_gcn.shape[1]
    e = pos_edge_index.shape[1]

    blk = 2048
    np_ = ((n + 1 + blk - 1) // blk) * blk
    pad_node = n

    grain2 = 12 * NW * CHUNK
    e2 = e + n
    e2p = ((e2 + grain2 - 1) // grain2) * grain2
    per_w2 = e2p // NW
    chunks2 = per_w2 // CHUNK

    grain6 = 2 * NW * CHUNK
    ea_n = 2 * e
    eap = ((ea_n + grain6 - 1) // grain6) * grain6
    per_w6 = eap // NW
    chunks6 = per_w6 // CHUNK

    x_p = jnp.zeros((np_, d), F32).at[:n].set(x)
    a2 = jnp.stack([a_src, a_dst], axis=1)
    loops = jnp.arange(n, dtype=I32)
    padtail2 = pad_node + (jnp.arange(e2p - e2, dtype=I32) % (np_ - n))
    srcp = jnp.concatenate([pos_edge_index[0], loops, padtail2])
    dstp = jnp.concatenate([pos_edge_index[1], loops, padtail2])
    padtail6 = jnp.arange(eap - ea_n, dtype=I32) % n
    eap_a = jnp.concatenate([pos_edge_index[0], neg_edge_index[0], padtail6])
    eap_b = jnp.concatenate([pos_edge_index[1], neg_edge_index[1], padtail6])
    src3 = srcp.reshape(NW, chunks2, CHUNK)
    dst4 = dstp.reshape(NW, chunks2 * 2, CHUNK // 2)
    z128 = jnp.zeros((np_, h_dim), F32)
    z64 = jnp.zeros((np_, o_dim), F32)
    bgat2 = b_gat.reshape(1, h_dim)
    bgcn2 = b_gcn.reshape(1, o_dim)

    nb = np_ // blk

    h, aa = pl.pallas_call(
        _k1_body,
        grid=(nb,),
        in_specs=[
            pl.BlockSpec((blk, d), lambda i: (i, 0)),
            pl.BlockSpec((d, h_dim), lambda i: (0, 0)),
            pl.BlockSpec((d, 2), lambda i: (0, 0)),
        ],
        out_specs=[
            pl.BlockSpec((blk, h_dim), lambda i: (i, 0)),
            pl.BlockSpec((blk, 2), lambda i: (i, 0)),
        ],
        out_shape=[
            jax.ShapeDtypeStruct((np_, h_dim), F32),
            jax.ShapeDtypeStruct((np_, 2), F32),
        ],
    )(x_p, W_gat, a2)

    ex_e, den_p, deg_p = pl.kernel(
        functools.partial(_gat_scalar_body, np_, per_w2),
        out_type=(
            jax.ShapeDtypeStruct((e2p,), F32),
            jax.ShapeDtypeStruct((NW, np_), F32),
            jax.ShapeDtypeStruct((NW, np_), F32),
        ),
        mesh=_sc_mesh(),
        compiler_params=_SC_PARAMS,
        scratch_types=[
            pltpu.VMEM((2 * np_,), F32),
            pltpu.VMEM((per_w2,), I32),
            pltpu.VMEM((per_w2,), I32),
            pltpu.VMEM((per_w2,), F32),
            pltpu.VMEM((np_,), F32),
            pltpu.VMEM((np_,), F32),
        ],
    )(aa.reshape(-1), srcp, dstp)
    den_t = den_p.T
    deg_t = deg_p.T

    acc1p = pl.kernel(
        functools.partial(_gat_row_body, np_, chunks2),
        out_type=jax.ShapeDtypeStruct((NC, np_, h_dim), F32),
        mesh=_sc_mesh(),
        compiler_params=_SC_PARAMS,
        scratch_types=[
            pltpu.VMEM((per_w2,), F32),
            pltpu.VMEM((2, CHUNK), I32),
            pltpu.VMEM((2, CHUNK), I32),
            pltpu.VMEM((4, CHUNK // 2), I32),
            pltpu.VMEM((4, CHUNK // 2), I32),
            pltpu.VMEM((CHUNK, h_dim), F32),
            pltpu.VMEM((CHUNK, h_dim), F32),
            pltpu.VMEM_SHARED((np_, h_dim), F32),
            pltpu.SemaphoreType.DMA,
            pltpu.SemaphoreType.DMA,
            pltpu.SemaphoreType.DMA,
        ],
    )(h, ex_e, src3, dst4, z128)

    hw2, dinv = pl.pallas_call(
        functools.partial(_k3_body, n),
        grid=(nb,),
        in_specs=[
            pl.BlockSpec((NC, blk, h_dim), lambda i: (0, i, 0)),
            pl.BlockSpec((blk, NW), lambda i: (i, 0)),
            pl.BlockSpec((blk, NW), lambda i: (i, 0)),
            pl.BlockSpec((1, h_dim), lambda i: (0, 0)),
            pl.BlockSpec((h_dim, o_dim), lambda i: (0, 0)),
        ],
        out_specs=[
            pl.BlockSpec((blk, o_dim), lambda i: (i, 0)),
            pl.BlockSpec((blk, 1), lambda i: (i, 0)),
        ],
        out_shape=[
            jax.ShapeDtypeStruct((np_, o_dim), F32),
            jax.ShapeDtypeStruct((np_, 1), F32),
        ],
    )(acc1p, den_t, deg_t, bgat2, W_gcn)

    acc2p = pl.kernel(
        functools.partial(_gcn_edge_body, np_, chunks2),
        out_type=jax.ShapeDtypeStruct((NC, np_, o_dim), F32),
        mesh=_sc_mesh(),
        compiler_params=_SC_PARAMS,
        scratch_types=[
            pltpu.VMEM((chunks2, CHUNK), I32),
            pltpu.VMEM((chunks2, CHUNK), I32),
            pltpu.VMEM((CHUNK, o_dim), F32),
            pltpu.VMEM((CHUNK, o_dim), F32),
            pltpu.VMEM((CHUNK, o_dim), F32),
            pltpu.VMEM_SHARED((np_, o_dim), F32),
            pltpu.VMEM_SHARED((np_, o_dim), F32),
            pltpu.SemaphoreType.DMA,
            pltpu.SemaphoreType.DMA,
            pltpu.SemaphoreType.DMA,
        ],
    )(hw2, src3, dstp.reshape(NW, chunks2, CHUNK), z64)

    z = pl.pallas_call(
        _k5_body,
        grid=(nb,),
        in_specs=[
            pl.BlockSpec((NC, blk, o_dim), lambda i: (0, i, 0)),
            pl.BlockSpec((blk, 1), lambda i: (i, 0)),
            pl.BlockSpec((1, o_dim), lambda i: (0, 0)),
        ],
        out_specs=pl.BlockSpec((blk, o_dim), lambda i: (i, 0)),
        out_shape=jax.ShapeDtypeStruct((np_, o_dim), F32),
    )(acc2p, dinv, bgcn2)

    logits_p = pl.kernel(
        functools.partial(_decode_body, np_, chunks6),
        out_type=jax.ShapeDtypeStruct((eap,), F32),
        mesh=_sc_mesh(),
        compiler_params=_SC_PARAMS,
        scratch_types=[
            pltpu.VMEM((per_w6,), I32),
            pltpu.VMEM((per_w6,), I32),
            pltpu.VMEM((CHUNK, o_dim), F32),
            pltpu.VMEM((CHUNK, o_dim), F32),
            pltpu.VMEM((CHUNK, o_dim), F32),
            pltpu.VMEM((CHUNK, o_dim), F32),
            pltpu.VMEM((2 * CHUNK,), F32),
            pltpu.VMEM_SHARED((np_, o_dim), F32),
            pltpu.SemaphoreType.DMA,
            pltpu.SemaphoreType.DMA,
        ],
    )(z, eap_a, eap_b)

    return logits_p[:ea_n]


# speedup vs baseline: 1.0526x; 1.0526x over previous
import functools

import jax
import jax.numpy as jnp
from jax import lax
from jax.experimental import pallas as pl
from jax.experimental.pallas import tpu as pltpu
from jax.experimental.pallas import tpu_sc as plsc

F32 = jnp.float32
I32 = jnp.int32

NC = 2
NS = 16
NW = NC * NS
L = 16

CHUNK = 128


def _sc_mesh():
    return plsc.VectorSubcoreMesh(
        core_axis_name="c", subcore_axis_name="s", num_cores=NC, num_subcores=NS
    )


_SC_PARAMS = pltpu.CompilerParams(
    needs_layout_passes=False, use_tc_tiling_on_sc=False
)


def _k1_body(x_ref, wg_ref, a2_ref, h_ref, aa_ref):
    h = jnp.dot(x_ref[...], wg_ref[...], preferred_element_type=F32)
    h_ref[...] = h
    aa_ref[...] = jnp.dot(h, a2_ref[...], preferred_element_type=F32)


def _gat_scalar_body(np_, pw, aa_hbm, src_hbm, dst_hbm, ex_out, den_out,
                     deg_out, aa_v, srcv, dstv, ex_v, den_v, deg_v):
    c = lax.axis_index("c")
    s = lax.axis_index("s")
    wid = c * NS + s

    pltpu.sync_copy(aa_hbm, aa_v)
    wbase = wid * pw
    pltpu.sync_copy(src_hbm.at[pl.ds(wbase, pw)], srcv)
    pltpu.sync_copy(dst_hbm.at[pl.ds(wbase, pw)], dstv)
    zf = jnp.zeros((L,), F32)

    def _zero(i, _):
        den_v[pl.ds(i * L, L)] = zf
        deg_v[pl.ds(i * L, L)] = zf
        return 0
    lax.fori_loop(0, np_ // L, _zero, 0)

    ones_f = jnp.full((L,), 1.0, F32)

    def _grp(g, _):
        sidx = srcv[pl.ds(g * L, L)]
        didx = dstv[pl.ds(g * L, L)]
        a_s = plsc.load_gather(aa_v, [sidx * 2])
        a_d = plsc.load_gather(aa_v, [didx * 2 + 1])
        e = a_s + a_d
        e = jnp.where(e > 0, e, e * 0.2)
        ex = jnp.exp(e)
        ex_v[pl.ds(g * L, L)] = ex
        plsc.addupdate_scatter(den_v, [didx], ex)
        plsc.addupdate_scatter(deg_v, [didx], ones_f)
        return 0

    lax.fori_loop(0, pw // L, _grp, 0)

    pltpu.sync_copy(ex_v, ex_out.at[pl.ds(wbase, pw)])
    pltpu.sync_copy(den_v, den_out.at[wid])
    pltpu.sync_copy(deg_v, deg_out.at[wid])


HC = CHUNK // 2


def _gat_row_body(np_, chunks, h_hbm, ex_hbm, src3_hbm, dst4_hbm, z128_hbm,
                  acc_out, ex_v, src0, src1, dst0, dst1, rows0, rows1,
                  acc_sh, semA, semB, semS):
    c = lax.axis_index("c")
    s = lax.axis_index("s")
    wid = c * NS + s
    rpt = np_ // NS

    pltpu.sync_copy(z128_hbm.at[pl.ds(s * rpt, rpt)], acc_sh.at[pl.ds(s * rpt, rpt)])
    plsc.subcore_barrier()

    wbase = wid * (chunks * CHUNK)
    pltpu.sync_copy(ex_hbm.at[pl.ds(wbase, chunks * CHUNK)], ex_v)

    def _stage(ci0, srcb, dstb):
        pltpu.sync_copy(src3_hbm.at[wid, pl.ds(ci0, 2)], srcb)
        pltpu.sync_copy(dst4_hbm.at[wid, pl.ds(ci0 * 2, 4)], dstb)

    def _scale_half(rows_v, ci, hh):
        base = ci * CHUNK + hh * HC

        def body(g, _):
            ex16 = ex_v[pl.ds(base + g * L, L)]
            for t in range(L):
                w = ex16.at[jnp.full((L,), t, I32)].get(
                    mode="promise_in_bounds")
                i = hh * HC + g * L + t
                for j in range(128 // L):
                    v = rows_v[i, pl.ds(j * L, L)]
                    rows_v[i, pl.ds(j * L, L)] = v * w
            return 0
        lax.fori_loop(0, HC // L, body, 0)

    def _proc(rows, semG, ci, dstb, jp):
        pltpu.make_async_copy(h_hbm.at[pl.ds(0, CHUNK)], rows, semG).wait()
        _scale_half(rows, ci, 0)
        pltpu.async_copy(rows.at[pl.ds(0, HC)], acc_sh.at[dstb.at[2 * jp]],
                         semS, add=True)
        _scale_half(rows, ci, 1)
        pltpu.async_copy(rows.at[pl.ds(HC, HC)], acc_sh.at[dstb.at[2 * jp + 1]],
                         semS, add=True)
        for _ in range(2):
            pltpu.make_async_copy(h_hbm.at[pl.ds(0, HC)],
                                  rows.at[pl.ds(0, HC)], semS).wait()

    _stage(0, src0, dst0)
    pltpu.async_copy(h_hbm.at[src0.at[0]], rows0, semA)

    def _quad(q, _):
        c0 = q * 4
        pltpu.async_copy(h_hbm.at[src0.at[1]], rows1, semB)
        _stage(c0 + 2, src1, dst1)

        _proc(rows0, semA, c0, dst0, 0)
        pltpu.async_copy(h_hbm.at[src1.at[0]], rows0, semA)

        _proc(rows1, semB, c0 + 1, dst0, 1)
        pltpu.async_copy(h_hbm.at[src1.at[1]], rows1, semB)

        @pl.when(c0 + 4 < chunks)
        def _():
            _stage(c0 + 4, src0, dst0)

        _proc(rows0, semA, c0 + 2, dst1, 0)

        @pl.when(c0 + 4 < chunks)
        def _():
            pltpu.async_copy(h_hbm.at[src0.at[0]], rows0, semA)

        _proc(rows1, semB, c0 + 3, dst1, 1)
        return 0

    lax.fori_loop(0, chunks // 4, _quad, 0)

    plsc.subcore_barrier()
    pltpu.sync_copy(acc_sh.at[pl.ds(s * rpt, rpt)], acc_out.at[c, pl.ds(s * rpt, rpt)])


def _k3_body(n, accp_ref, den_ref, deg_ref, bgat_ref, wgcn_ref, hw2_ref,
             dinv_ref):
    srow = accp_ref[0] + accp_ref[1]
    den = jnp.sum(den_ref[...], axis=1, keepdims=True)
    deg = jnp.sum(deg_ref[...], axis=1, keepdims=True)
    h1 = jnp.tanh(srow / (den + 1e-16) + bgat_ref[...])
    hw = jnp.dot(h1, wgcn_ref[...], preferred_element_type=F32)
    blk = hw.shape[0]
    rowi = pl.program_id(0) * blk + lax.broadcasted_iota(I32, (blk, 1), 0)
    dinv = jnp.where((deg > 0) & (rowi < n), lax.rsqrt(deg), 0.0)
    hw2_ref[...] = hw * dinv
    dinv_ref[...] = dinv


def _gcn_edge_body(np_, chunks, hw2_hbm, src3_hbm, dst3_hbm, z64_hbm, acc_out,
                   srcv, dstv, rows0, rows1, rows2, acc_sh, semA, semB, semC):
    c = lax.axis_index("c")
    s = lax.axis_index("s")
    wid = c * NS + s
    rpt = np_ // NS

    pltpu.sync_copy(z64_hbm.at[pl.ds(s * rpt, rpt)], acc_sh.at[pl.ds(s * rpt, rpt)])
    plsc.subcore_barrier()

    pltpu.sync_copy(src3_hbm.at[wid], srcv)
    pltpu.sync_copy(dst3_hbm.at[wid], dstv)

    bufs = (rows0, rows1, rows2)
    sems = (semA, semB, semC)
    for k in range(3):
        pltpu.async_copy(hw2_hbm.at[srcv.at[k]], bufs[k], sems[k])

    def _trip(q, _):
        for k in range(3):
            ci = q * 3 + k
            pltpu.make_async_copy(hw2_hbm.at[pl.ds(0, CHUNK)], bufs[k], sems[k]).wait()
            pltpu.sync_copy(bufs[k], acc_sh.at[dstv.at[ci]], add=True)

            @pl.when(ci + 3 < chunks)
            def _():
                pltpu.async_copy(hw2_hbm.at[srcv.at[ci + 3]], bufs[k], sems[k])
        return 0

    lax.fori_loop(0, chunks // 3, _trip, 0)

    plsc.subcore_barrier()
    pltpu.sync_copy(acc_sh.at[pl.ds(s * rpt, rpt)], acc_out.at[c, pl.ds(s * rpt, rpt)])


def _k5_body(acc2p_ref, dinv_ref, bgcn_ref, z_ref):
    z_ref[...] = dinv_ref[...] * (acc2p_ref[0] + acc2p_ref[1]) + bgcn_ref[...]


def _decode_body(np_, chunks, z_hbm, ea_hbm, eb_hbm, out_hbm, eav, ebv,
                 za0, zb0, za1, zb1, dots_v, z_sh, semA, semB):
    c = lax.axis_index("c")
    s = lax.axis_index("s")
    wid = c * NS + s
    iota = lax.iota(I32, L)
    wbase = wid * (chunks * CHUNK)
    rpt = np_ // NS

    pltpu.sync_copy(z_hbm.at[pl.ds(s * rpt, rpt)], z_sh.at[pl.ds(s * rpt, rpt)])
    pltpu.sync_copy(ea_hbm.at[pl.ds(wbase, chunks * CHUNK)], eav)
    pltpu.sync_copy(eb_hbm.at[pl.ds(wbase, chunks * CHUNK)], ebv)
    plsc.subcore_barrier()

    def _dots(za, zb, k):
        def _grp(g, _):
            rows = g * L + iota
            acc = jnp.zeros((L,), F32)
            for j in range(64):
                col = (iota + j) & 63
                acc = acc + (plsc.load_gather(za, [rows, col])
                             * plsc.load_gather(zb, [rows, col]))
            dots_v[pl.ds(k * CHUNK + g * L, L)] = acc
            return 0
        lax.fori_loop(0, CHUNK // L, _grp, 0)

    def _fire(ci, za, zb, sem):
        pltpu.async_copy(z_sh.at[eav.at[pl.ds(ci * CHUNK, CHUNK)]], za, sem)
        pltpu.async_copy(z_sh.at[ebv.at[pl.ds(ci * CHUNK, CHUNK)]], zb, sem)

    def _drain(za, zb, sem):
        pltpu.make_async_copy(z_hbm.at[pl.ds(0, CHUNK)], za, sem).wait()
        pltpu.make_async_copy(z_hbm.at[pl.ds(0, CHUNK)], zb, sem).wait()

    _fire(0, za0, zb0, semA)

    def _pair(p, _):
        ci0 = p * 2
        _fire(ci0 + 1, za1, zb1, semB)

        _drain(za0, zb0, semA)
        _dots(za0, zb0, 0)

        @pl.when(ci0 + 2 < chunks)
        def _():
            _fire(ci0 + 2, za0, zb0, semA)

        _drain(za1, zb1, semB)
        _dots(za1, zb1, 1)
        pltpu.sync_copy(dots_v, out_hbm.at[pl.ds(wbase + ci0 * CHUNK, 2 * CHUNK)])
        return 0

    lax.fori_loop(0, chunks // 2, _pair, 0)


def kernel(x, pos_edge_index, neg_edge_index, W_gat, a_src, a_dst, b_gat,
           W_gcn, b_gcn):
    n, d = x.shape
    h_dim = W_gat.shape[1]
    o_dim = W_gcn.shape[1]
    e = pos_edge_index.shape[1]

    blk = 2048
    np_ = ((n + 1 + blk - 1) // blk) * blk
    pad_node = n

    grain2 = 12 * NW * CHUNK
    e2 = e + n
    e2p = ((e2 + grain2 - 1) // grain2) * grain2
    per_w2 = e2p // NW
    chunks2 = per_w2 // CHUNK

    grain6 = 2 * NW * CHUNK
    ea_n = 2 * e
    eap = ((ea_n + grain6 - 1) // grain6) * grain6
    per_w6 = eap // NW
    chunks6 = per_w6 // CHUNK

    x_p = jnp.zeros((np_, d), F32).at[:n].set(x)
    a2 = jnp.stack([a_src, a_dst], axis=1)
    loops = jnp.arange(n, dtype=I32)
    padtail2 = pad_node + (jnp.arange(e2p - e2, dtype=I32) % (np_ - n))
    srcp = jnp.concatenate([pos_edge_index[0], loops, padtail2])
    dstp = jnp.concatenate([pos_edge_index[1], loops, padtail2])
    padtail6 = jnp.arange(eap - ea_n, dtype=I32) % n
    eap_a = jnp.concatenate([pos_edge_index[0], neg_edge_index[0], padtail6])
    eap_b = jnp.concatenate([pos_edge_index[1], neg_edge_index[1], padtail6])
    src3 = srcp.reshape(NW, chunks2, CHUNK)
    dst4 = dstp.reshape(NW, chunks2 * 2, CHUNK // 2)
    z128 = jnp.zeros((np_, h_dim), F32)
    z64 = jnp.zeros((np_, o_dim), F32)
    bgat2 = b_gat.reshape(1, h_dim)
    bgcn2 = b_gcn.reshape(1, o_dim)

    nb = np_ // blk

    h, aa = pl.pallas_call(
        _k1_body,
        grid=(nb,),
        in_specs=[
            pl.BlockSpec((blk, d), lambda i: (i, 0)),
            pl.BlockSpec((d, h_dim), lambda i: (0, 0)),
            pl.BlockSpec((d, 2), lambda i: (0, 0)),
        ],
        out_specs=[
            pl.BlockSpec((blk, h_dim), lambda i: (i, 0)),
            pl.BlockSpec((blk, 2), lambda i: (i, 0)),
        ],
        out_shape=[
            jax.ShapeDtypeStruct((np_, h_dim), F32),
            jax.ShapeDtypeStruct((np_, 2), F32),
        ],
    )(x_p, W_gat, a2)

    ex_e, den_p, deg_p = pl.kernel(
        functools.partial(_gat_scalar_body, np_, per_w2),
        out_type=(
            jax.ShapeDtypeStruct((e2p,), F32),
            jax.ShapeDtypeStruct((NW, np_), F32),
            jax.ShapeDtypeStruct((NW, np_), F32),
        ),
        mesh=_sc_mesh(),
        compiler_params=_SC_PARAMS,
        scratch_types=[
            pltpu.VMEM((2 * np_,), F32),
            pltpu.VMEM((per_w2,), I32),
            pltpu.VMEM((per_w2,), I32),
            pltpu.VMEM((per_w2,), F32),
            pltpu.VMEM((np_,), F32),
            pltpu.VMEM((np_,), F32),
        ],
    )(aa.reshape(-1), srcp, dstp)
    den_t = den_p.T
    deg_t = deg_p.T

    acc1p = pl.kernel(
        functools.partial(_gat_row_body, np_, chunks2),
        out_type=jax.ShapeDtypeStruct((NC, np_, h_dim), F32),
        mesh=_sc_mesh(),
        compiler_params=_SC_PARAMS,
        scratch_types=[
            pltpu.VMEM((per_w2,), F32),
            pltpu.VMEM((2, CHUNK), I32),
            pltpu.VMEM((2, CHUNK), I32),
            pltpu.VMEM((4, CHUNK // 2), I32),
            pltpu.VMEM((4, CHUNK // 2), I32),
            pltpu.VMEM((CHUNK, h_dim), F32),
            pltpu.VMEM((CHUNK, h_dim), F32),
            pltpu.VMEM_SHARED((np_, h_dim), F32),
            pltpu.SemaphoreType.DMA,
            pltpu.SemaphoreType.DMA,
            pltpu.SemaphoreType.DMA,
        ],
    )(h, ex_e, src3, dst4, z128)

    hw2, dinv = pl.pallas_call(
        functools.partial(_k3_body, n),
        grid=(nb,),
        in_specs=[
            pl.BlockSpec((NC, blk, h_dim), lambda i: (0, i, 0)),
            pl.BlockSpec((blk, NW), lambda i: (i, 0)),
            pl.BlockSpec((blk, NW), lambda i: (i, 0)),
            pl.BlockSpec((1, h_dim), lambda i: (0, 0)),
            pl.BlockSpec((h_dim, o_dim), lambda i: (0, 0)),
        ],
        out_specs=[
            pl.BlockSpec((blk, o_dim), lambda i: (i, 0)),
            pl.BlockSpec((blk, 1), lambda i: (i, 0)),
        ],
        out_shape=[
            jax.ShapeDtypeStruct((np_, o_dim), F32),
            jax.ShapeDtypeStruct((np_, 1), F32),
        ],
    )(acc1p, den_t, deg_t, bgat2, W_gcn)

    acc2p = pl.kernel(
        functools.partial(_gcn_edge_body, np_, chunks2),
        out_type=jax.ShapeDtypeStruct((NC, np_, o_dim), F32),
        mesh=_sc_mesh(),
        compiler_params=_SC_PARAMS,
        scratch_types=[
            pltpu.VMEM((chunks2, CHUNK), I32),
            pltpu.VMEM((chunks2, CHUNK), I32),
            pltpu.VMEM((CHUNK, o_dim), F32),
            pltpu.VMEM((CHUNK, o_dim), F32),
            pltpu.VMEM((CHUNK, o_dim), F32),
            pltpu.VMEM_SHARED((np_, o_dim), F32),
            pltpu.SemaphoreType.DMA,
            pltpu.SemaphoreType.DMA,
            pltpu.SemaphoreType.DMA,
        ],
    )(hw2, src3, dstp.reshape(NW, chunks2, CHUNK), z64)

    z = pl.pallas_call(
        _k5_body,
        grid=(nb,),
        in_specs=[
            pl.BlockSpec((NC, blk, o_dim), lambda i: (0, i, 0)),
            pl.BlockSpec((blk, 1), lambda i: (i, 0)),
            pl.BlockSpec((1, o_dim), lambda i: (0, 0)),
        ],
        out_specs=pl.BlockSpec((blk, o_dim), lambda i: (i, 0)),
        out_shape=jax.ShapeDtypeStruct((np_, o_dim), F32),
    )(acc2p, dinv, bgcn2)

    logits_p = pl.kernel(
        functools.partial(_decode_body, np_, chunks6),
        out_type=jax.ShapeDtypeStruct((eap,), F32),
        mesh=_sc_mesh(),
        compiler_params=_SC_PARAMS,
        scratch_types=[
            pltpu.VMEM((per_w6,), I32),
            pltpu.VMEM((per_w6,), I32),
            pltpu.VMEM((CHUNK, o_dim), F32),
            pltpu.VMEM((CHUNK, o_dim), F32),
            pltpu.VMEM((CHUNK, o_dim), F32),
            pltpu.VMEM((CHUNK, o_dim), F32),
            pltpu.VMEM((2 * CHUNK,), F32),
            pltpu.VMEM_SHARED((np_, o_dim), F32),
            pltpu.SemaphoreType.DMA,
            pltpu.SemaphoreType.DMA,
        ],
    )(z, eap_a, eap_b)

    return logits_p[:ea_n]
